# baseline (speedup 1.0000x reference)
import jax
import jax.numpy as jnp
from jax import lax
from jax.experimental import pallas as pl
from jax.experimental.pallas import tpu as pltpu
from jax.experimental.pallas import tpu_sc as plsc

B = 16
M = 16384
C = 512
N = 2048
XP = 8
HALF = 1024
L = 16

_SIZES = [(1 << k) for k in range(10, -1, -1)]


def _sc_body(bidx0, feats0, xyzp0, bidx1, feats1, xyzp1, zf, zp,
             f0, p0, m0, f1, p1, m1,
             bv0, bv1, mb0, mb1, sem):
    c = lax.axis_index("c")
    s = lax.axis_index("s")
    wid = c * 16 + s
    b = wid // 2
    h = wid % 2
    d0 = wid * HALF

    cp0 = pltpu.async_copy(bidx0, bv0, sem)
    cp1 = pltpu.async_copy(bidx1, bv1, sem)
    cp0.wait()
    cp1.wait()

    iota = lax.broadcasted_iota(jnp.int32, (L,), 0)

    def counts(bv):
        def step(i, carry):
            lo, hi = carry
            x = bv[pl.ds(i * L, L)]
            lo = lo + jnp.where(x < b, 1, 0).astype(jnp.int32)
            hi = hi + jnp.where(x <= b, 1, 0).astype(jnp.int32)
            return lo, hi
        z = jnp.zeros((L,), jnp.int32)
        lo, hi = lax.fori_loop(0, M // L, step, (z, z))
        return jnp.sum(lo), jnp.sum(hi)

    seg0 = counts(bv0)
    seg1 = counts(bv1)

    def valid_rows(seg):
        s0, e0 = seg
        v = jnp.clip(e0 - s0 - h * HALF, 0, HALF)
        return v, s0 + h * HALF

    def fill_mask(mbuf, v):
        def step(i, carry):
            idx = i * L + iota
            mbuf[pl.ds(i * L, L)] = jnp.where(idx < v, 0.0, 1.0).astype(jnp.float32)
            return carry
        lax.fori_loop(0, HALF // L, step, 0)

    srcs = (
        (seg0, feats0, xyzp0, f0, p0, m0, mb0),
        (seg1, feats1, xyzp1, f1, p1, m1, mb1),
    )

    def emit(src, start):
        seg, feats, xyzp, fout, pout, mout, mbuf = src
        v, srow = valid_rows(seg)
        nv = HALF - v
        for size in _SIZES:
            k = size.bit_length()
            off = (v >> k) << k
            zoff = (nv >> k) << k

            @pl.when((v & size) != 0)
            def _(off=off, size=size):
                for src_ref, dst_ref in ((feats, fout), (xyzp, pout)):
                    d = pltpu.make_async_copy(
                        src_ref.at[pl.ds(srow + off, size)],
                        dst_ref.at[pl.ds(d0 + off, size)], sem)
                    d.start() if start else d.wait()

            @pl.when((nv & size) != 0)
            def _(zoff=zoff, size=size):
                for src_ref, dst_ref in ((zf, fout), (zp, pout)):
                    d = pltpu.make_async_copy(
                        src_ref.at[pl.ds(0, size)],
                        dst_ref.at[pl.ds(d0 + v + zoff, size)], sem)
                    d.start() if start else d.wait()

        d = pltpu.make_async_copy(mbuf, mout.at[pl.ds(d0, HALF)], sem)
        d.start() if start else d.wait()

    for src in srcs:
        fill_mask(src[6], valid_rows(src[0])[0])
    for src in srcs:
        emit(src, True)
    for src in srcs:
        emit(src, False)


def _build(interpret=False):
    mesh = plsc.VectorSubcoreMesh(core_axis_name="c", subcore_axis_name="s",
                                  num_cores=2, num_subcores=16)
    out_type = (
        jax.ShapeDtypeStruct((B * N, C), jnp.float32),
        jax.ShapeDtypeStruct((B * N, XP), jnp.float32),
        jax.ShapeDtypeStruct((B * N,), jnp.float32),
        jax.ShapeDtypeStruct((B * N, C), jnp.float32),
        jax.ShapeDtypeStruct((B * N, XP), jnp.float32),
        jax.ShapeDtypeStruct((B * N,), jnp.float32),
    )
    scratch = [
        pltpu.VMEM((M,), jnp.int32),
        pltpu.VMEM((M,), jnp.int32),
        pltpu.VMEM((HALF,), jnp.float32),
        pltpu.VMEM((HALF,), jnp.float32),
        pltpu.SemaphoreType.DMA,
    ]
    return pl.kernel(_sc_body, out_type=out_type, mesh=mesh,
                     scratch_types=scratch, interpret=interpret,
                     compiler_params=pltpu.CompilerParams(
                         use_tc_tiling_on_sc=False,
                         needs_layout_passes=False))


def kernel(bidx_src0, xyz_src0, feats_src0, bidx_src1, xyz_src1, feats_src1,
           batch_size, interpret=False):
    del batch_size
    xyzp0 = jnp.pad(xyz_src0, ((0, 0), (0, XP - 3)))
    xyzp1 = jnp.pad(xyz_src1, ((0, 0), (0, XP - 3)))
    zf = jnp.zeros((HALF, C), jnp.float32)
    zp = jnp.zeros((HALF, XP), jnp.float32)
    fn = _build(interpret)
    f0, p0, m0, f1, p1, m1 = fn(bidx_src0, feats_src0, xyzp0,
                                bidx_src1, feats_src1, xyzp1, zf, zp)
    return (f0.reshape(B, N, C), p0.reshape(B, N, XP)[:, :, :3],
            m0.reshape(B, N),
            f1.reshape(B, N, C), p1.reshape(B, N, XP)[:, :, :3],
            m1.reshape(B, N))


# speedup vs baseline: 11.5614x; 11.5614x over previous
import jax
import jax.numpy as jnp
from jax import lax
from jax.experimental import pallas as pl
from jax.experimental.pallas import tpu as pltpu
from jax.experimental.pallas import tpu_sc as plsc

B = 16
M = 16384
C = 512
N = 2048
XP = 8
HALF = 1024
L = 16
CH = 64
NCH = HALF // CH
ZR = 32

_SIZES_FULL = [(1 << k) for k in range(10, -1, -1)]
_SIZES_REM = [(1 << k) for k in range(5, -1, -1)]
_SIZES_ZTAIL = [(1 << k) for k in range(4, -1, -1)]


def _sc_body(bidx0, feats0, xyzp0, bidx1, feats1, xyzp1, zf, zp,
             f0, p0, m0, f1, p1, m1,
             bbuf, ring, xybuf, mbuf, zbuf,
             sem_in, semo0, semo1, sem_x, sem_misc):
    cax = lax.axis_index("c")
    sax = lax.axis_index("s")
    wid = cax * 16 + sax
    b = wid // 2
    h = wid % 2
    d0 = wid * HALF
    semos = (semo0, semo1)

    cz = pltpu.async_copy(zf, zbuf, sem_misc)
    cb = pltpu.async_copy(bidx0, bbuf, sem_misc)
    cz.wait()
    cb.wait()

    iota = lax.broadcasted_iota(jnp.int32, (L,), 0)

    def counts(bv):
        def step(i, carry):
            lo, hi = carry
            for u in range(4):
                x = bv[pl.ds((i * 4 + u) * L, L)]
                lo = lo + jnp.where(x < b, 1, 0).astype(jnp.int32)
                hi = hi + jnp.where(x <= b, 1, 0).astype(jnp.int32)
            return lo, hi
        z = jnp.zeros((L,), jnp.int32)
        lo, hi = lax.fori_loop(0, M // L // 4, step, (z, z))
        return jnp.sum(lo), jnp.sum(hi)

    seg0 = counts(bbuf)
    pltpu.sync_copy(bidx1, bbuf)
    seg1 = counts(bbuf)

    def fill_mask(v):
        def step(i, carry):
            for u in range(4):
                j = i * 4 + u
                idx = j * L + iota
                mbuf[pl.ds(j * L, L)] = jnp.where(
                    idx < v, 0.0, 1.0).astype(jnp.float32)
            return carry
        lax.fori_loop(0, HALF // L // 4, step, 0)

    deferred = []

    for si, (seg, feats, xyzp, fout, pout, mout) in enumerate((
            (seg0, feats0, xyzp0, f0, p0, m0),
            (seg1, feats1, xyzp1, f1, p1, m1))):
        s0, e0 = seg
        v = jnp.clip(e0 - s0 - h * HALF, 0, HALF)
        srow = s0 + h * HALF
        rem = v & (CH - 1)
        nv = HALF - v
        xyb = xybuf.at[pl.ds(si * HALF, HALF)]

        def xyz_ins(action, v=v, nv=nv, srow=srow, xyb=xyb, xyzp=xyzp):
            for size in _SIZES_FULL:
                k = size.bit_length()
                off = (v >> k) << k
                zoff = (nv >> k) << k

                @pl.when((v & size) != 0)
                def _(off=off, size=size):
                    d = pltpu.make_async_copy(
                        xyzp.at[pl.ds(srow + off, size)],
                        xyb.at[pl.ds(off, size)], sem_x)
                    d.start() if action == "s" else d.wait()

                @pl.when((nv & size) != 0)
                def _(zoff=zoff, size=size):
                    d = pltpu.make_async_copy(
                        zp.at[pl.ds(0, size)],
                        xyb.at[pl.ds(v + zoff, size)], sem_x)
                    d.start() if action == "s" else d.wait()
        xyz_ins("s")

        for c in range(NCH):
            @pl.when(v >= (c + 1) * CH)
            def _(c=c):
                slot = c & 1
                if c >= 2:
                    pltpu.make_async_copy(
                        ring.at[pl.ds(slot * CH, CH)],
                        fout.at[pl.ds(d0 + (c - 2) * CH, CH)],
                        semos[slot]).wait()
                pltpu.async_copy(
                    feats.at[pl.ds(srow + c * CH, CH)],
                    ring.at[pl.ds(slot * CH, CH)], sem_in).wait()
                pltpu.async_copy(
                    ring.at[pl.ds(slot * CH, CH)],
                    fout.at[pl.ds(d0 + c * CH, CH)], semos[slot])
        for slot in range(2):
            @pl.when(v >= (slot + 1) * CH)
            def _(slot=slot):
                pltpu.make_async_copy(
                    ring.at[pl.ds(slot * CH, CH)],
                    fout.at[pl.ds(d0, CH)], semos[slot]).wait()

        @pl.when(rem != 0)
        def _(v=v, rem=rem, srow=srow, fout=fout):
            base = srow + (v - rem)
            for action in ("s", "w"):
                for size in _SIZES_REM:
                    k = size.bit_length()
                    off = (rem >> k) << k

                    @pl.when((rem & size) != 0)
                    def _(off=off, size=size):
                        d = pltpu.make_async_copy(
                            feats.at[pl.ds(base + off, size)],
                            ring.at[pl.ds(off, size)], sem_in)
                        d.start() if action == "s" else d.wait()
            pltpu.sync_copy(ring.at[pl.ds(0, CH)],
                            fout.at[pl.ds(d0 + v - rem, CH)])

        def zfill(action, v=v, nv=nv, fout=fout):
            q = nv >> 5

            def zstep(j, carry):
                d = pltpu.make_async_copy(
                    zbuf, fout.at[pl.ds(d0 + v + j * ZR, ZR)], sem_misc)
                d.start() if action == "s" else d.wait()
                return carry
            lax.fori_loop(0, q, zstep, 0)
            tail = nv & (ZR - 1)
            toff = nv - tail
            for size in _SIZES_ZTAIL:
                k = size.bit_length()
                off = (tail >> k) << k

                @pl.when((tail & size) != 0)
                def _(off=off, size=size):
                    d = pltpu.make_async_copy(
                        zbuf.at[pl.ds(0, size)],
                        fout.at[pl.ds(d0 + v + toff + off, size)], sem_misc)
                    d.start() if action == "s" else d.wait()
        zfill("s")
        deferred.append(lambda zfill=zfill: zfill("w"))

        xyz_ins("w")
        pltpu.async_copy(xyb, pout.at[pl.ds(d0, HALF)], sem_misc)
        deferred.append(lambda xyb=xyb, pout=pout: pltpu.make_async_copy(
            xyb, pout.at[pl.ds(d0, HALF)], sem_misc).wait())

        fill_mask(v)
        pltpu.sync_copy(mbuf, mout.at[pl.ds(d0, HALF)])

    for wait_fn in deferred:
        wait_fn()


def _build(interpret=False):
    mesh = plsc.VectorSubcoreMesh(core_axis_name="c", subcore_axis_name="s",
                                  num_cores=2, num_subcores=16)
    out_type = (
        jax.ShapeDtypeStruct((B * N, C), jnp.float32),
        jax.ShapeDtypeStruct((B * N, XP), jnp.float32),
        jax.ShapeDtypeStruct((B * N,), jnp.float32),
        jax.ShapeDtypeStruct((B * N, C), jnp.float32),
        jax.ShapeDtypeStruct((B * N, XP), jnp.float32),
        jax.ShapeDtypeStruct((B * N,), jnp.float32),
    )
    scratch = [
        pltpu.VMEM((M,), jnp.int32),
        pltpu.VMEM((2 * CH, C), jnp.float32),
        pltpu.VMEM((2 * HALF, XP), jnp.float32),
        pltpu.VMEM((HALF,), jnp.float32),
        pltpu.VMEM((ZR, C), jnp.float32),
        pltpu.SemaphoreType.DMA,
        pltpu.SemaphoreType.DMA,
        pltpu.SemaphoreType.DMA,
        pltpu.SemaphoreType.DMA,
        pltpu.SemaphoreType.DMA,
    ]
    return pl.kernel(_sc_body, out_type=out_type, mesh=mesh,
                     scratch_types=scratch, interpret=interpret,
                     compiler_params=pltpu.CompilerParams(
                         use_tc_tiling_on_sc=False,
                         needs_layout_passes=False))


def kernel(bidx_src0, xyz_src0, feats_src0, bidx_src1, xyz_src1, feats_src1,
           batch_size, interpret=False):
    del batch_size
    xyzp0 = jnp.pad(xyz_src0, ((0, 0), (0, XP - 3)))
    xyzp1 = jnp.pad(xyz_src1, ((0, 0), (0, XP - 3)))
    zf = jnp.zeros((ZR, C), jnp.float32)
    zp = jnp.zeros((HALF, XP), jnp.float32)
    fn = _build(interpret)
    f0, p0, m0, f1, p1, m1 = fn(bidx_src0, feats_src0, xyzp0,
                                bidx_src1, feats_src1, xyzp1, zf, zp)
    return (f0.reshape(B, N, C), p0.reshape(B, N, XP)[:, :, :3],
            m0.reshape(B, N),
            f1.reshape(B, N, C), p1.reshape(B, N, XP)[:, :, :3],
            m1.reshape(B, N))


# speedup vs baseline: 11.8117x; 1.0216x over previous
import jax
import jax.numpy as jnp
from jax import lax
from jax.experimental import pallas as pl
from jax.experimental.pallas import tpu as pltpu
from jax.experimental.pallas import tpu_sc as plsc

B = 16
M = 16384
C = 512
N = 2048
XP = 8
HALF = 1024
L = 16
CH = 64
NCH = HALF // CH
ZR = 32

_SIZES_FULL = [(1 << k) for k in range(10, -1, -1)]
_SIZES_REM = [(1 << k) for k in range(5, -1, -1)]
_SIZES_ZTAIL = [(1 << k) for k in range(4, -1, -1)]


XF = 3 * HALF
XT = XF + 64


def _sc_body(bidx0, feats0, xyzf0, bidx1, feats1, xyzf1, zf,
             f0, p0, m0, f1, p1, m1,
             bbuf, ring, xybuf, xytmp, mbuf, zbuf,
             sem_in, semo0, semo1, sem_x, sem_misc):
    cax = lax.axis_index("c")
    sax = lax.axis_index("s")
    wid = cax * 16 + sax
    b = wid // 2
    h = wid % 2
    d0 = wid * HALF
    semos = (semo0, semo1)

    cz = pltpu.async_copy(zf, zbuf, sem_misc)
    cb = pltpu.async_copy(bidx0, bbuf, sem_misc)
    cz.wait()
    cb.wait()

    iota = lax.broadcasted_iota(jnp.int32, (L,), 0)

    def counts(bv):
        def step(i, carry):
            lo, hi = carry
            for u in range(4):
                x = bv[pl.ds((i * 4 + u) * L, L)]
                lo = lo + jnp.where(x < b, 1, 0).astype(jnp.int32)
                hi = hi + jnp.where(x <= b, 1, 0).astype(jnp.int32)
            return lo, hi
        z = jnp.zeros((L,), jnp.int32)
        lo, hi = lax.fori_loop(0, M // L // 4, step, (z, z))
        return jnp.sum(lo), jnp.sum(hi)

    seg0 = counts(bbuf)
    pltpu.sync_copy(bidx1, bbuf)
    seg1 = counts(bbuf)

    def fill_mask(v):
        def step(i, carry):
            for u in range(4):
                j = i * 4 + u
                idx = j * L + iota
                mbuf[pl.ds(j * L, L)] = jnp.where(
                    idx < v, 0.0, 1.0).astype(jnp.float32)
            return carry
        lax.fori_loop(0, HALF // L // 4, step, 0)

    deferred = []

    for si, (seg, feats, xyzf, fout, pout, mout) in enumerate((
            (seg0, feats0, xyzf0, f0, p0, m0),
            (seg1, feats1, xyzf1, f1, p1, m1))):
        s0, e0 = seg
        v = jnp.clip(e0 - s0 - h * HALF, 0, HALF)
        srow = s0 + h * HALF
        rem = v & (CH - 1)
        nv = HALF - v
        xbase = si * XT

        vstart = srow * 3
        a0 = jnp.clip((vstart >> 3) << 3, 0, 3 * M - XT)
        a0 = pl.multiple_of(a0, 8)
        sh = vstart - a0
        xyz_in = pltpu.make_async_copy(
            xyzf.at[pl.ds(a0, XT)], xytmp.at[pl.ds(0, XT)], sem_x)
        xyz_in.start()

        for c in range(NCH):
            @pl.when(v >= (c + 1) * CH)
            def _(c=c):
                slot = c & 1
                if c >= 2:
                    pltpu.make_async_copy(
                        ring.at[pl.ds(slot * CH, CH)],
                        fout.at[pl.ds(d0 + (c - 2) * CH, CH)],
                        semos[slot]).wait()
                pltpu.async_copy(
                    feats.at[pl.ds(srow + c * CH, CH)],
                    ring.at[pl.ds(slot * CH, CH)], sem_in).wait()
                pltpu.async_copy(
                    ring.at[pl.ds(slot * CH, CH)],
                    fout.at[pl.ds(d0 + c * CH, CH)], semos[slot])
        for slot in range(2):
            @pl.when(v >= (slot + 1) * CH)
            def _(slot=slot):
                pltpu.make_async_copy(
                    ring.at[pl.ds(slot * CH, CH)],
                    fout.at[pl.ds(d0, CH)], semos[slot]).wait()

        @pl.when(rem != 0)
        def _(v=v, rem=rem, srow=srow, fout=fout):
            base = srow + (v - rem)
            tz = CH - rem
            for action in ("s", "w"):
                for size in _SIZES_REM:
                    k = size.bit_length()
                    off = (rem >> k) << k
                    zoff = (tz >> k) << k

                    @pl.when((rem & size) != 0)
                    def _(off=off, size=size):
                        d = pltpu.make_async_copy(
                            feats.at[pl.ds(base + off, size)],
                            ring.at[pl.ds(off, size)], sem_in)
                        d.start() if action == "s" else d.wait()

                    @pl.when((tz & size) != 0)
                    def _(zoff=zoff, size=size):
                        d = pltpu.make_async_copy(
                            zf.at[pl.ds(0, size)],
                            ring.at[pl.ds(rem + zoff, size)], sem_in)
                        d.start() if action == "s" else d.wait()
            pltpu.sync_copy(ring.at[pl.ds(0, CH)],
                            fout.at[pl.ds(d0 + v - rem, CH)])

        def zfill(action, v=v, fout=fout):
            zstart = ((v + CH - 1) >> 6) << 6
            q = (HALF - zstart) >> 5

            def zstep(j, carry):
                d = pltpu.make_async_copy(
                    zbuf, fout.at[pl.ds(d0 + zstart + j * ZR, ZR)], sem_misc)
                d.start() if action == "s" else d.wait()
                return carry
            lax.fori_loop(0, q, zstep, 0)
        zfill("s")
        deferred.append(lambda zfill=zfill: zfill("w"))

        xyz_in.wait()
        vlen = v * 3

        def xstep(i, carry):
            xybuf[pl.ds(xbase + i * L, L)] = xytmp[pl.ds(sh + i * L, L)]
            return carry
        lax.fori_loop(0, vlen >> 4, xstep, 0)
        fl = (vlen >> 4) << 4
        xv = xytmp[pl.ds(sh + fl, L)]
        xybuf[pl.ds(xbase + fl, L)] = jnp.where(
            iota < (vlen & 15), xv, 0.0).astype(jnp.float32)

        def xzstep(j, carry):
            xybuf[pl.ds(xbase + j * L, L)] = jnp.zeros((L,), jnp.float32)
            return carry
        lax.fori_loop((vlen + 15) >> 4, XF // L, xzstep, 0)
        p0off = pl.multiple_of(d0 * 3, 8)
        pltpu.async_copy(xybuf.at[pl.ds(xbase, XF)],
                         pout.at[pl.ds(p0off, XF)], sem_misc)
        deferred.append(
            lambda xbase=xbase, pout=pout, p0off=p0off: pltpu.make_async_copy(
                xybuf.at[pl.ds(xbase, XF)],
                pout.at[pl.ds(p0off, XF)], sem_misc).wait())

        fill_mask(v)
        pltpu.sync_copy(mbuf, mout.at[pl.ds(pl.multiple_of(d0, 8), HALF)])

    for wait_fn in deferred:
        wait_fn()


def _build(interpret=False):
    mesh = plsc.VectorSubcoreMesh(core_axis_name="c", subcore_axis_name="s",
                                  num_cores=2, num_subcores=16)
    out_type = (
        jax.ShapeDtypeStruct((B * N, C), jnp.float32),
        jax.ShapeDtypeStruct((B * N * 3,), jnp.float32),
        jax.ShapeDtypeStruct((B * N,), jnp.float32),
        jax.ShapeDtypeStruct((B * N, C), jnp.float32),
        jax.ShapeDtypeStruct((B * N * 3,), jnp.float32),
        jax.ShapeDtypeStruct((B * N,), jnp.float32),
    )
    scratch = [
        pltpu.VMEM((M,), jnp.int32),
        pltpu.VMEM((2 * CH, C), jnp.float32),
        pltpu.VMEM((2 * XT,), jnp.float32),
        pltpu.VMEM((XT + L,), jnp.float32),
        pltpu.VMEM((HALF,), jnp.float32),
        pltpu.VMEM((ZR, C), jnp.float32),
        pltpu.SemaphoreType.DMA,
        pltpu.SemaphoreType.DMA,
        pltpu.SemaphoreType.DMA,
        pltpu.SemaphoreType.DMA,
        pltpu.SemaphoreType.DMA,
    ]
    return pl.kernel(_sc_body, out_type=out_type, mesh=mesh,
                     scratch_types=scratch, interpret=interpret,
                     compiler_params=pltpu.CompilerParams(
                         use_tc_tiling_on_sc=False,
                         needs_layout_passes=False))


def kernel(bidx_src0, xyz_src0, feats_src0, bidx_src1, xyz_src1, feats_src1,
           batch_size, interpret=False):
    del batch_size
    zf = jnp.zeros((ZR, C), jnp.float32)
    fn = _build(interpret)
    f0, p0, m0, f1, p1, m1 = fn(bidx_src0, feats_src0, xyz_src0.reshape(-1),
                                bidx_src1, feats_src1, xyz_src1.reshape(-1),
                                zf)
    return (f0.reshape(B, N, C), p0.reshape(B, N, 3), m0.reshape(B, N),
            f1.reshape(B, N, C), p1.reshape(B, N, 3), m1.reshape(B, N))


# speedup vs baseline: 15.4735x; 1.3100x over previous
import jax
import jax.numpy as jnp
from jax import lax
from jax.experimental import pallas as pl
from jax.experimental.pallas import tpu as pltpu
from jax.experimental.pallas import tpu_sc as plsc

B = 16
M = 16384
C = 512
N = 2048
HALF = 1024
L = 16
CH = 64
NCH = HALF // CH
TR = 4 * CH
XFETCH = 1040
XSTRIDE = 1056

_SIZES_REM = [(1 << k) for k in range(5, -1, -1)]


def _sc_body(bidx0, ftab0, xyzt0, bidx1, ftab1, xyzt1, zf,
             f0, p0, m0, f1, p1, m1,
             bbuf, ring, ixbuf, xyb, xytmp, mbuf, zbuf,
             sem_in, semo0, semo1, sem_x, sem_misc):
    cax = lax.axis_index("c")
    sax = lax.axis_index("s")
    wid = cax * 16 + sax
    b = wid // 2
    h = wid % 2
    d0 = wid * HALF
    semos = (semo0, semo1)

    cz = pltpu.async_copy(zf, zbuf, sem_misc)
    cb = pltpu.async_copy(bidx0, bbuf, sem_misc)
    cz.wait()
    cb.wait()

    iota = lax.broadcasted_iota(jnp.int32, (L,), 0)

    def counts(bv):
        def step(i, carry):
            lo, hi = carry
            for u in range(4):
                x = bv[pl.ds((i * 4 + u) * L, L)]
                lo = lo + jnp.where(x < b, 1, 0).astype(jnp.int32)
                hi = hi + jnp.where(x <= b, 1, 0).astype(jnp.int32)
            return lo, hi
        z = jnp.zeros((L,), jnp.int32)
        lo, hi = lax.fori_loop(0, M // L // 4, step, (z, z))
        return jnp.sum(lo), jnp.sum(hi)

    seg0 = counts(bbuf)
    pltpu.sync_copy(bidx1, bbuf)
    seg1 = counts(bbuf)

    def fill_mask(v, mb):
        def step(i, carry):
            for u in range(4):
                j = i * 4 + u
                idx = j * L + iota
                mbuf[pl.ds(mb + j * L, L)] = jnp.where(
                    idx < v, 0.0, 1.0).astype(jnp.float32)
            return carry
        lax.fori_loop(0, HALF // L // 4, step, 0)

    def tiled_idx(i, g16):
        return ((i >> 3) << 5) + (g16 << 3) + (i & 7)

    deferred = []

    for si, (seg, ftab, xyzt, fout, pout, mout) in enumerate((
            (seg0, ftab0, xyzt0, f0, p0, m0),
            (seg1, ftab1, xyzt1, f1, p1, m1))):
        s0, e0 = seg
        v = jnp.clip(e0 - s0 - h * HALF, 0, HALF)
        srow = s0 + h * HALF
        rem = v & (CH - 1)
        nv = HALF - v

        sal = jnp.minimum((srow >> 3) << 3, M - XFETCH)
        sh = srow - sal
        xyz_ins = []
        for comp in range(3):
            a0 = pl.multiple_of(comp * M + sal, 8)
            d = pltpu.make_async_copy(
                xyzt.at[pl.ds(a0, XFETCH)],
                xytmp.at[pl.ds(comp * XSTRIDE, XFETCH)], sem_x)
            d.start()
            xyz_ins.append(d)

        seeds = []
        for t in range(16):
            j = t * L + iota
            seeds.append(tiled_idx(srow + (j >> 2), j & 3))

        def fire_chunk(c, slot):
            for t in range(16):
                ixbuf[pl.ds(slot * TR + t * L, L)] = seeds[t] + c * TR
            gs = []
            for hc in range(2):
                gs.append(pltpu.async_copy(
                    ftab.at[ixbuf.at[pl.ds(slot * TR + hc * 128, 128)]],
                    ring.at[pl.ds(slot * TR + hc * 128, 128)], sem_in))
            return gs

        for c in range(NCH):
            @pl.when(v >= (c + 1) * CH)
            def _(c=c):
                slot = c & 1
                if c >= 2:
                    pltpu.make_async_copy(
                        ring.at[pl.ds(slot * TR, TR)],
                        fout.at[pl.ds((d0 + (c - 2) * CH) * 4, TR)],
                        semos[slot]).wait()
                for g in fire_chunk(c, slot):
                    g.wait()
                pltpu.async_copy(
                    ring.at[pl.ds(slot * TR, TR)],
                    fout.at[pl.ds((d0 + c * CH) * 4, TR)], semos[slot])
        for slot in range(2):
            @pl.when(v >= (slot + 1) * CH)
            def _(slot=slot):
                pltpu.make_async_copy(
                    ring.at[pl.ds(slot * TR, TR)],
                    fout.at[pl.ds(d0 * 4, TR)], semos[slot]).wait()

        @pl.when(rem != 0)
        def _(v=v, rem=rem, srow=srow, fout=fout, ftab=ftab):
            cb0 = v - rem
            imax = srow + v - 1
            for t in range(16):
                j = t * L + iota
                i = jnp.minimum(srow + cb0 + (j >> 2), imax)
                ixbuf[pl.ds(t * L, L)] = tiled_idx(i, j & 3)
            for hc in range(2):
                pltpu.async_copy(
                    ftab.at[ixbuf.at[pl.ds(hc * 128, 128)]],
                    ring.at[pl.ds(hc * 128, 128)], sem_in).wait()
            tz = CH - rem
            for action in ("s", "w"):
                for size in _SIZES_REM:
                    k = size.bit_length()
                    zoff = (tz >> k) << k

                    @pl.when((tz & size) != 0)
                    def _(zoff=zoff, size=size):
                        d = pltpu.make_async_copy(
                            zf.at[pl.ds(0, 4 * size)],
                            ring.at[pl.ds(4 * (rem + zoff), 4 * size)],
                            sem_in)
                        d.start() if action == "s" else d.wait()
            pltpu.sync_copy(ring.at[pl.ds(0, TR)],
                            fout.at[pl.ds((d0 + cb0) * 4, TR)])

        def zfill(action, v=v, fout=fout):
            zstart = ((v + CH - 1) >> 6) << 6
            q = (HALF - zstart) >> 5

            def zstep(j, carry):
                d = pltpu.make_async_copy(
                    zbuf, fout.at[pl.ds((d0 + zstart) * 4 + j * 128, 128)],
                    sem_misc)
                d.start() if action == "s" else d.wait()
                return carry
            lax.fori_loop(0, q, zstep, 0)
        zfill("s")
        deferred.append(lambda zfill=zfill: zfill("w"))

        for d in xyz_ins:
            d.wait()
        for comp in range(3):
            tb = comp * XSTRIDE
            ob = (si * 3 + comp) * XFETCH

            def xstep(i, carry, tb=tb, ob=ob):
                xyb[pl.ds(ob + i * L, L)] = xytmp[pl.ds(tb + sh + i * L, L)]
                return carry
            lax.fori_loop(0, v >> 4, xstep, 0)
            fl = (v >> 4) << 4
            xv = xytmp[pl.ds(tb + sh + fl, L)]
            xyb[pl.ds(ob + fl, L)] = jnp.where(
                iota < (v & 15), xv, 0.0).astype(jnp.float32)

            def xzstep(i, carry, ob=ob):
                xyb[pl.ds(ob + i * L, L)] = jnp.zeros((L,), jnp.float32)
                return carry
            lax.fori_loop((v + 15) >> 4, HALF // L, xzstep, 0)
            pltpu.async_copy(
                xyb.at[pl.ds(ob, HALF)],
                pout.at[pl.ds(pl.multiple_of(comp * B * N + d0, 8), HALF)],
                sem_misc)
            deferred.append(
                lambda ob=ob, comp=comp, pout=pout: pltpu.make_async_copy(
                    xyb.at[pl.ds(ob, HALF)],
                    pout.at[pl.ds(pl.multiple_of(comp * B * N + d0, 8), HALF)],
                    sem_misc).wait())

        mb = si * HALF
        fill_mask(v, mb)
        for cg in range(8):
            moff = pl.multiple_of(
                (b >> 3) * (16 * 1024) + (h * 8 + cg) * 1024 + (b & 7) * 128, 8)
            pltpu.async_copy(mbuf.at[pl.ds(mb + cg * 128, 128)],
                             mout.at[pl.ds(moff, 128)], sem_misc)
            deferred.append(
                lambda mb=mb, cg=cg, moff=moff, mout=mout:
                pltpu.make_async_copy(
                    mbuf.at[pl.ds(mb + cg * 128, 128)],
                    mout.at[pl.ds(moff, 128)], sem_misc).wait())

    for wait_fn in deferred:
        wait_fn()


def _build(interpret=False):
    mesh = plsc.VectorSubcoreMesh(core_axis_name="c", subcore_axis_name="s",
                                  num_cores=2, num_subcores=16)
    out_type = (
        jax.ShapeDtypeStruct((B * N * 4, 128), jnp.float32),
        jax.ShapeDtypeStruct((3 * B * N,), jnp.float32),
        jax.ShapeDtypeStruct((B * N,), jnp.float32),
        jax.ShapeDtypeStruct((B * N * 4, 128), jnp.float32),
        jax.ShapeDtypeStruct((3 * B * N,), jnp.float32),
        jax.ShapeDtypeStruct((B * N,), jnp.float32),
    )
    scratch = [
        pltpu.VMEM((M,), jnp.int32),
        pltpu.VMEM((2 * TR, 128), jnp.float32),
        pltpu.VMEM((2 * TR,), jnp.int32),
        pltpu.VMEM((6 * XFETCH,), jnp.float32),
        pltpu.VMEM((3 * XSTRIDE,), jnp.float32),
        pltpu.VMEM((2 * HALF,), jnp.float32),
        pltpu.VMEM((128, 128), jnp.float32),
        pltpu.SemaphoreType.DMA,
        pltpu.SemaphoreType.DMA,
        pltpu.SemaphoreType.DMA,
        pltpu.SemaphoreType.DMA,
        pltpu.SemaphoreType.DMA,
    ]
    return pl.kernel(_sc_body, out_type=out_type, mesh=mesh,
                     scratch_types=scratch, interpret=interpret,
                     compiler_params=pltpu.CompilerParams(
                         use_tc_tiling_on_sc=False,
                         needs_layout_passes=False))


def kernel(bidx_src0, xyz_src0, feats_src0, bidx_src1, xyz_src1, feats_src1,
           batch_size, interpret=False):
    del batch_size
    zf = jnp.zeros((128, 128), jnp.float32)
    fn = _build(interpret)

    def ftab(feats):
        return feats.reshape(M // 8, 8, 4, 128).transpose(0, 2, 1, 3) \
                    .reshape(M // 8 * 32, 128)

    f0, p0, m0, f1, p1, m1 = fn(
        bidx_src0, ftab(feats_src0), xyz_src0.T.reshape(-1),
        bidx_src1, ftab(feats_src1), xyz_src1.T.reshape(-1), zf)

    def unmask(m):
        return m.reshape(2, 16, 8, 128).transpose(0, 2, 1, 3).reshape(B, N)

    def unpts(p):
        return p.reshape(3, B, N).transpose(1, 2, 0)

    return (f0.reshape(B, N, C), unpts(p0), unmask(m0),
            f1.reshape(B, N, C), unpts(p1), unmask(m1))
